# baseline (speedup 1.0000x reference)
import jax
import jax.numpy as jnp
from jax.experimental import pallas as pl

_NUM_USERS = 1_000_000
_NUM_SUBGROUPS = 1_000
_NUM_CHOICES = 100_000
_D = 64

_GRID = 125
_UB = _NUM_USERS // _GRID
_CB = _NUM_CHOICES // _GRID
_SB = _NUM_SUBGROUPS // _GRID


def _tables_kernel(sub_ref, cho_ref, user_ref, sub_out, cho_out, user_out):
    sub_out[...] = sub_ref[...]
    cho_out[...] = cho_ref[...]
    user_out[...] = jnp.broadcast_to(user_ref[...], (_UB, _D))


def kernel(sub_w, cho_w, user_token):
    sub_o, cho_o, user_o = pl.pallas_call(
        _tables_kernel,
        grid=(_GRID,),
        in_specs=[
            pl.BlockSpec((_SB, _D), lambda i: (i, 0)),
            pl.BlockSpec((_CB, _D), lambda i: (i, 0)),
            pl.BlockSpec((1, _D), lambda i: (0, 0)),
        ],
        out_specs=[
            pl.BlockSpec((_SB, _D), lambda i: (i, 0)),
            pl.BlockSpec((_CB, _D), lambda i: (i, 0)),
            pl.BlockSpec((_UB, _D), lambda i: (i, 0)),
        ],
        out_shape=[
            jax.ShapeDtypeStruct((_NUM_SUBGROUPS, _D), jnp.float32),
            jax.ShapeDtypeStruct((_NUM_CHOICES, _D), jnp.float32),
            jax.ShapeDtypeStruct((_NUM_USERS, _D), jnp.float32),
        ],
    )(sub_w, cho_w, user_token)
    return (sub_o, cho_o, user_o)
